# baseline (speedup 1.0000x reference)
import functools

import jax
import jax.numpy as jnp
from jax import lax
from jax.experimental import pallas as pl
from jax.experimental.pallas import tpu as pltpu

_VMEM_LIMIT = 48 * 1024 * 1024


def _fps_kernel(pc_ref, sel_ref, mind_ref, *, n, n_new, nsub):
    bb = pc_ref.shape[1]
    hb = bb // nsub
    colf = lax.broadcasted_iota(jnp.int32, (hb, n), 1).astype(jnp.float32)
    ncol = lax.broadcasted_iota(jnp.int32, (hb, n_new), 1)
    bigf = jnp.float32(3e9)
    mind_ref[...] = jnp.full((bb, n), jnp.inf, jnp.float32)
    sel_ref[...] = jnp.zeros((bb, n_new), jnp.int32)
    halves = []
    for h in range(nsub):
        sl = slice(h * hb, (h + 1) * hb)
        halves.append((pc_ref[0, sl], pc_ref[1, sl], pc_ref[2, sl]))

    def body(i, carry):
        out = []
        for h, (lx, ly, lz) in enumerate(carry):
            px, py, pz = halves[h]
            dx = px - lx
            dy = py - ly
            dz = pz - lz
            d = (dx * dx + dy * dy) + dz * dz
            nmin = jnp.minimum(mind_ref[pl.ds(h * hb, hb), :], d)
            mind_ref[pl.ds(h * hb, hb), :] = nmin
            m = jnp.max(nmin, axis=-1, keepdims=True)
            nxtf = jnp.min(jnp.where(nmin == m, colf, bigf), axis=-1, keepdims=True)
            sel_ref[pl.ds(h * hb, hb), :] = jnp.where(
                ncol == i, nxtf.astype(jnp.int32), sel_ref[pl.ds(h * hb, hb), :])
            sm = colf == nxtf
            nlx = jnp.sum(jnp.where(sm, px, 0.0), axis=-1, keepdims=True)
            nly = jnp.sum(jnp.where(sm, py, 0.0), axis=-1, keepdims=True)
            nlz = jnp.sum(jnp.where(sm, pz, 0.0), axis=-1, keepdims=True)
            out.append((nlx, nly, nlz))
        return tuple(out)

    init = tuple((px[:, 0:1], py[:, 0:1], pz[:, 0:1]) for px, py, pz in halves)
    lax.fori_loop(1, n_new, body, init)


def _fps(p, n_new):
    B, n, _ = p.shape
    ncores = 2 if B % 2 == 0 and (B // 2) % 8 == 0 else 1
    bb = B // ncores
    nsub = 2 if bb % 16 == 0 else 1
    pc = jnp.moveaxis(p, -1, 0)
    sel = pl.pallas_call(
        functools.partial(_fps_kernel, n=n, n_new=n_new, nsub=nsub),
        out_shape=jax.ShapeDtypeStruct((B, n_new), jnp.int32),
        grid_spec=pltpu.PrefetchScalarGridSpec(
            num_scalar_prefetch=0,
            grid=(ncores,),
            in_specs=[pl.BlockSpec((3, bb, n), lambda i: (0, i, 0))],
            out_specs=pl.BlockSpec((bb, n_new), lambda i: (i, 0)),
            scratch_shapes=[pltpu.VMEM((bb, n), jnp.float32)],
        ),
        compiler_params=pltpu.CompilerParams(
            dimension_semantics=("parallel",), vmem_limit_bytes=_VMEM_LIMIT),
    )(pc)
    return jnp.sort(sel, axis=-1)


def _knn_kernel(ps_ref, pt_ref, idx_ref, *, n, k):
    ps = ps_ref[0]
    pt = pt_ref[0]
    tq = ps.shape[0]
    cross = jnp.dot(ps, pt, preferred_element_type=jnp.float32)
    ps2 = jnp.sum(ps * ps, axis=-1, keepdims=True)
    p2 = jnp.sum(pt * pt, axis=0, keepdims=True)
    d2 = ps2 + p2 - 2.0 * cross

    colf = lax.broadcasted_iota(jnp.int32, (tq, n), 1).astype(jnp.float32)
    lanek = lax.broadcasted_iota(jnp.int32, (tq, k), 1)
    big = jnp.float32(3e9)
    acc = jnp.zeros((tq, k), jnp.float32)
    for t in range(k):
        m = jnp.min(d2, axis=-1, keepdims=True)
        sel = jnp.min(jnp.where(d2 == m, colf, big), axis=-1, keepdims=True)
        acc = jnp.where(lanek == t, sel, acc)
        d2 = jnp.where(colf == sel, jnp.inf, d2)
    idx_ref[0] = acc.astype(jnp.int32)


def _knn(p_sub, p, k):
    B, n_new, _ = p_sub.shape
    _, n, _ = p.shape
    tq = min(256, n_new)
    ps = jnp.pad(p_sub, ((0, 0), (0, 0), (0, 5)))
    pt = jnp.pad(jnp.swapaxes(p, 1, 2), ((0, 0), (0, 5), (0, 0)))
    return pl.pallas_call(
        functools.partial(_knn_kernel, n=n, k=k),
        out_shape=jax.ShapeDtypeStruct((B, n_new, k), jnp.int32),
        grid_spec=pltpu.PrefetchScalarGridSpec(
            num_scalar_prefetch=0,
            grid=(B, n_new // tq),
            in_specs=[
                pl.BlockSpec((1, tq, 8), lambda b, i: (b, i, 0)),
                pl.BlockSpec((1, 8, n), lambda b, i: (b, 0, 0)),
            ],
            out_specs=pl.BlockSpec((1, tq, k), lambda b, i: (b, i, 0)),
        ),
        compiler_params=pltpu.CompilerParams(
            dimension_semantics=("parallel", "parallel"),
            vmem_limit_bytes=_VMEM_LIMIT),
    )(ps, pt)


def _mgp_kernel(x_ref, w_ref, b_ref, base_ref, sh_ref, z_ref, y_s, *, tq, k):
    @pl.when(pl.program_id(1) == 0)
    def _mlp():
        y = jnp.dot(x_ref[0], w_ref[...], preferred_element_type=jnp.float32)
        y_s[...] = jnp.maximum(y + b_ref[...], 0.0)

    iota8 = lax.broadcasted_iota(jnp.int32, (8, y_s.shape[-1]), 0)

    def group(g, _):
        base = g * 8
        pooled = []
        for qq in range(8):
            q = base + qq
            ws = []
            for j in range(k):
                c = y_s[pl.ds(pl.multiple_of(base_ref[0, q, j], 8), 8), :]
                ws.append(pltpu.roll(c, sh_ref[0, q, j], axis=0))
            while len(ws) > 1:
                ws = [jnp.maximum(a, b) for a, b in zip(ws[::2], ws[1::2])]
            pooled.append(ws[0])
        lvl = [jnp.where(iota8 == 2 * i, pooled[2 * i], pooled[2 * i + 1])
               for i in range(4)]
        lvl = [jnp.where(iota8 < 4 * i + 2, lvl[2 * i], lvl[2 * i + 1])
               for i in range(2)]
        z_ref[0, pl.ds(base, 8), :] = jnp.where(iota8 < 4, lvl[0], lvl[1])
        return 0

    lax.fori_loop(0, tq // 8, group, 0)


def _mlp_gather_pool(x, w, bias, idx, k):
    B, n, d_in = x.shape
    d_out = w.shape[1]
    n_new = idx.shape[1]
    tq = min(256, n_new)
    b2 = bias.reshape(1, d_out)
    chunk_base = idx & ~jnp.int32(7)
    qsub = (jnp.arange(n_new, dtype=jnp.int32) % 8)[None, :, None]
    shift = qsub - (idx & 7)
    return pl.pallas_call(
        functools.partial(_mgp_kernel, tq=tq, k=k),
        out_shape=jax.ShapeDtypeStruct((B, n_new, d_out), jnp.float32),
        grid_spec=pltpu.PrefetchScalarGridSpec(
            num_scalar_prefetch=0,
            grid=(B, n_new // tq),
            in_specs=[
                pl.BlockSpec((1, n, d_in), lambda b, i: (b, 0, 0)),
                pl.BlockSpec((d_in, d_out), lambda b, i: (0, 0)),
                pl.BlockSpec((1, d_out), lambda b, i: (0, 0)),
                pl.BlockSpec((1, tq, k), lambda b, i: (b, i, 0),
                             memory_space=pltpu.SMEM),
                pl.BlockSpec((1, tq, k), lambda b, i: (b, i, 0),
                             memory_space=pltpu.SMEM),
            ],
            out_specs=pl.BlockSpec((1, tq, d_out), lambda b, i: (b, i, 0)),
            scratch_shapes=[pltpu.VMEM((n, d_out), jnp.float32)],
        ),
        compiler_params=pltpu.CompilerParams(
            dimension_semantics=("parallel", "arbitrary"),
            vmem_limit_bytes=_VMEM_LIMIT),
    )(x, w, b2, chunk_base, shift)


def kernel(x, p, w, bias, *, factor=2, knn_k=8):
    B, n, d_in = x.shape
    n_new = -(-n // factor)

    sub_idx = _fps(p, n_new)
    p_sub = jnp.take_along_axis(p, sub_idx[..., None], axis=1)
    knn_idx = _knn(p_sub, p, knn_k)
    z = _mlp_gather_pool(x, w, bias, knn_idx, knn_k)
    return z, p_sub, knn_idx


# speedup vs baseline: 1.2348x; 1.2348x over previous
import functools

import jax
import jax.numpy as jnp
from jax import lax
from jax.experimental import pallas as pl
from jax.experimental.pallas import tpu as pltpu

_VMEM_LIMIT = 64 * 1024 * 1024


def _fps_kernel(pc_ref, sel_ref, m0_ref, m1_ref, s0_ref, s1_ref, *, n, n_new, nsub):
    bb = pc_ref.shape[1]
    hb = bb // nsub
    colf = lax.broadcasted_iota(jnp.int32, (hb, n), 1).astype(jnp.float32)
    ncol = lax.broadcasted_iota(jnp.int32, (hb, n_new), 1)
    bigf = jnp.float32(3e9)
    mind = (m0_ref, m1_ref)[:nsub]
    sels = (s0_ref, s1_ref)[:nsub]
    for h in range(nsub):
        mind[h][...] = jnp.full((hb, n), jnp.inf, jnp.float32)
        sels[h][...] = jnp.zeros((hb, n_new), jnp.int32)

    def body(i, carry):
        out = []
        for h, (lx, ly, lz) in enumerate(carry):
            sl = slice(h * hb, (h + 1) * hb)
            px = pc_ref[0, sl]
            py = pc_ref[1, sl]
            pz = pc_ref[2, sl]
            dx = px - lx
            dy = py - ly
            dz = pz - lz
            d = (dx * dx + dy * dy) + dz * dz
            nmin = jnp.minimum(mind[h][...], d)
            mind[h][...] = nmin
            m = jnp.max(nmin, axis=-1, keepdims=True)
            nxtf = jnp.min(jnp.where(nmin == m, colf, bigf), axis=-1, keepdims=True)
            sels[h][...] = jnp.where(ncol == i, nxtf.astype(jnp.int32), sels[h][...])
            sm = colf == nxtf
            nlx = jnp.sum(jnp.where(sm, px, 0.0), axis=-1, keepdims=True)
            nly = jnp.sum(jnp.where(sm, py, 0.0), axis=-1, keepdims=True)
            nlz = jnp.sum(jnp.where(sm, pz, 0.0), axis=-1, keepdims=True)
            out.append((nlx, nly, nlz))
        return tuple(out)

    init = tuple(
        (pc_ref[0, h * hb:h * hb + hb, 0:1],
         pc_ref[1, h * hb:h * hb + hb, 0:1],
         pc_ref[2, h * hb:h * hb + hb, 0:1])
        for h in range(nsub))
    lax.fori_loop(1, n_new, body, init)
    for h in range(nsub):
        sel_ref[pl.ds(h * hb, hb), :] = sels[h][...]


def _fps(p, n_new):
    B, n, _ = p.shape
    ncores = 2 if B % 2 == 0 and (B // 2) % 8 == 0 else 1
    bb = B // ncores
    nsub = 2 if bb % 16 == 0 else 1
    hb = bb // nsub
    pc = jnp.moveaxis(p, -1, 0)
    sel = pl.pallas_call(
        functools.partial(_fps_kernel, n=n, n_new=n_new, nsub=nsub),
        out_shape=jax.ShapeDtypeStruct((B, n_new), jnp.int32),
        grid_spec=pltpu.PrefetchScalarGridSpec(
            num_scalar_prefetch=0,
            grid=(ncores,),
            in_specs=[pl.BlockSpec((3, bb, n), lambda i: (0, i, 0))],
            out_specs=pl.BlockSpec((bb, n_new), lambda i: (i, 0)),
            scratch_shapes=[pltpu.VMEM((hb, n), jnp.float32),
                            pltpu.VMEM((hb, n), jnp.float32),
                            pltpu.VMEM((hb, n_new), jnp.int32),
                            pltpu.VMEM((hb, n_new), jnp.int32)],
        ),
        compiler_params=pltpu.CompilerParams(
            dimension_semantics=("parallel",), vmem_limit_bytes=_VMEM_LIMIT),
    )(pc)
    return jnp.sort(sel, axis=-1)


def _knn_kernel(ps_ref, pt_ref, idx_ref, *, n, k):
    ps = ps_ref[0]
    pt = pt_ref[0]
    tq = ps.shape[0]
    cross = jnp.dot(ps, pt, preferred_element_type=jnp.float32)
    ps2 = jnp.sum(ps * ps, axis=-1, keepdims=True)
    p2 = jnp.sum(pt * pt, axis=0, keepdims=True)
    d2 = ps2 + p2 - 2.0 * cross

    colf = lax.broadcasted_iota(jnp.int32, (tq, n), 1).astype(jnp.float32)
    lanek = lax.broadcasted_iota(jnp.int32, (tq, k), 1)
    big = jnp.float32(3e9)
    acc = jnp.zeros((tq, k), jnp.float32)
    for t in range(k):
        m = jnp.min(d2, axis=-1, keepdims=True)
        sel = jnp.min(jnp.where(d2 == m, colf, big), axis=-1, keepdims=True)
        acc = jnp.where(lanek == t, sel, acc)
        d2 = jnp.where(colf == sel, jnp.inf, d2)
    idx_ref[0] = acc.astype(jnp.int32)


def _knn(p_sub, p, k):
    B, n_new, _ = p_sub.shape
    _, n, _ = p.shape
    tq = min(256, n_new)
    ps = jnp.pad(p_sub, ((0, 0), (0, 0), (0, 5)))
    pt = jnp.pad(jnp.swapaxes(p, 1, 2), ((0, 0), (0, 5), (0, 0)))
    return pl.pallas_call(
        functools.partial(_knn_kernel, n=n, k=k),
        out_shape=jax.ShapeDtypeStruct((B, n_new, k), jnp.int32),
        grid_spec=pltpu.PrefetchScalarGridSpec(
            num_scalar_prefetch=0,
            grid=(B, n_new // tq),
            in_specs=[
                pl.BlockSpec((1, tq, 8), lambda b, i: (b, i, 0)),
                pl.BlockSpec((1, 8, n), lambda b, i: (b, 0, 0)),
            ],
            out_specs=pl.BlockSpec((1, tq, k), lambda b, i: (b, i, 0)),
        ),
        compiler_params=pltpu.CompilerParams(
            dimension_semantics=("parallel", "parallel"),
            vmem_limit_bytes=_VMEM_LIMIT),
    )(ps, pt)


def _mlp_kernel(x_ref, w_ref, b_ref, y_ref):
    y = jnp.dot(x_ref[...], w_ref[...], preferred_element_type=jnp.float32)
    y_ref[...] = jnp.maximum(y + b_ref[...], 0.0)


def _mlp(x_flat, w, bias, *, tm=2048):
    M, d_in = x_flat.shape
    d_out = w.shape[1]
    tm = min(tm, M)
    b2 = bias.reshape(1, d_out)
    return pl.pallas_call(
        _mlp_kernel,
        out_shape=jax.ShapeDtypeStruct((M, d_out), jnp.float32),
        grid_spec=pltpu.PrefetchScalarGridSpec(
            num_scalar_prefetch=0,
            grid=(M // tm,),
            in_specs=[
                pl.BlockSpec((tm, d_in), lambda i: (i, 0)),
                pl.BlockSpec((d_in, d_out), lambda i: (0, 0)),
                pl.BlockSpec((1, d_out), lambda i: (0, 0)),
            ],
            out_specs=pl.BlockSpec((tm, d_out), lambda i: (i, 0)),
        ),
        compiler_params=pltpu.CompilerParams(
            dimension_semantics=("parallel",), vmem_limit_bytes=_VMEM_LIMIT),
    )(x_flat, w, b2)


def _gpool_kernel(y_ref, idx_ref, z_ref, *, tq, k, unroll):
    def run(u, _):
        for qq in range(unroll):
            q = u * unroll + qq
            rows = [y_ref[idx_ref[0, q, j], 0] for j in range(k)]
            while len(rows) > 1:
                rows = [jnp.maximum(a, b) for a, b in zip(rows[::2], rows[1::2])]
            z_ref[q, 0] = rows[0]
        return 0

    lax.fori_loop(0, tq // unroll, run, 0)


def _gather_pool(y3, idx, k, n, n_new, d_out):
    B = idx.shape[0]
    tq = min(256, n_new)
    nblk = n_new // tq
    return pl.pallas_call(
        functools.partial(_gpool_kernel, tq=tq, k=k, unroll=8),
        out_shape=jax.ShapeDtypeStruct((B * n_new, 1, d_out), jnp.float32),
        grid_spec=pltpu.PrefetchScalarGridSpec(
            num_scalar_prefetch=0,
            grid=(B, nblk),
            in_specs=[
                pl.BlockSpec((n, 1, d_out), lambda b, i: (b, 0, 0)),
                pl.BlockSpec((1, tq, k), lambda b, i: (b, i, 0),
                             memory_space=pltpu.SMEM),
            ],
            out_specs=pl.BlockSpec(
                (tq, 1, d_out), lambda b, i, _nblk=nblk: (b * _nblk + i, 0, 0)),
        ),
        compiler_params=pltpu.CompilerParams(
            dimension_semantics=("parallel", "arbitrary"),
            vmem_limit_bytes=_VMEM_LIMIT),
    )(y3, idx)


def kernel(x, p, w, bias, *, factor=2, knn_k=8):
    B, n, d_in = x.shape
    d_out = w.shape[1]
    n_new = -(-n // factor)

    sub_idx = _fps(p, n_new)
    p_sub = jnp.take_along_axis(p, sub_idx[..., None], axis=1)
    knn_idx = _knn(p_sub, p, knn_k)
    y = _mlp(x.reshape(B * n, d_in), w, bias)
    z3 = _gather_pool(y.reshape(B * n, 1, d_out), knn_idx,
                      knn_k, n, n_new, d_out)
    z = z3.reshape(B, n_new, d_out)
    return z, p_sub, knn_idx


# speedup vs baseline: 1.2378x; 1.0024x over previous
import functools

import jax
import jax.numpy as jnp
from jax import lax
from jax.experimental import pallas as pl
from jax.experimental.pallas import tpu as pltpu

_VMEM_LIMIT = 64 * 1024 * 1024


def _fps_kernel(pc_ref, sel_ref, m0_ref, m1_ref, s0_ref, s1_ref, *, n, n_new, nsub):
    bb = pc_ref.shape[1]
    hb = bb // nsub
    colf = lax.broadcasted_iota(jnp.int32, (hb, n), 1).astype(jnp.float32)
    ncol = lax.broadcasted_iota(jnp.int32, (hb, n_new), 1)
    bigf = jnp.float32(3e9)
    mind = (m0_ref, m1_ref)[:nsub]
    sels = (s0_ref, s1_ref)[:nsub]
    for h in range(nsub):
        mind[h][...] = jnp.full((hb, n), jnp.inf, jnp.float32)
        sels[h][...] = jnp.zeros((hb, n_new), jnp.int32)

    def body(i, carry):
        ps, nmins, mss, nxtfs, out = [], [], [], [], []
        for h, (lx, ly, lz) in enumerate(carry):
            sl = slice(h * hb, (h + 1) * hb)
            px = pc_ref[0, sl]
            py = pc_ref[1, sl]
            pz = pc_ref[2, sl]
            ps.append((px, py, pz))
            dx = px - lx
            dy = py - ly
            dz = pz - lz
            d = (dx * dx + dy * dy) + dz * dz
            nmin = jnp.minimum(mind[h][...], d)
            mind[h][...] = nmin
            nmins.append(nmin)
        for h in range(nsub):
            mss.append(jnp.max(nmins[h], axis=-1, keepdims=True))
        for h in range(nsub):
            nxtfs.append(jnp.min(jnp.where(nmins[h] == mss[h], colf, bigf),
                                 axis=-1, keepdims=True))
        for h in range(nsub):
            sels[h][...] = jnp.where(
                ncol == i, nxtfs[h].astype(jnp.int32), sels[h][...])
            px, py, pz = ps[h]
            sm = colf == nxtfs[h]
            nlx = jnp.sum(jnp.where(sm, px, 0.0), axis=-1, keepdims=True)
            nly = jnp.sum(jnp.where(sm, py, 0.0), axis=-1, keepdims=True)
            nlz = jnp.sum(jnp.where(sm, pz, 0.0), axis=-1, keepdims=True)
            out.append((nlx, nly, nlz))
        return tuple(out)

    init = tuple(
        (pc_ref[0, h * hb:h * hb + hb, 0:1],
         pc_ref[1, h * hb:h * hb + hb, 0:1],
         pc_ref[2, h * hb:h * hb + hb, 0:1])
        for h in range(nsub))
    lax.fori_loop(1, n_new, body, init)
    for h in range(nsub):
        sel_ref[pl.ds(h * hb, hb), :] = sels[h][...]


def _fps(p, n_new):
    B, n, _ = p.shape
    ncores = 2 if B % 2 == 0 and (B // 2) % 8 == 0 else 1
    bb = B // ncores
    nsub = 2 if bb % 16 == 0 else 1
    hb = bb // nsub
    pc = jnp.moveaxis(p, -1, 0)
    sel = pl.pallas_call(
        functools.partial(_fps_kernel, n=n, n_new=n_new, nsub=nsub),
        out_shape=jax.ShapeDtypeStruct((B, n_new), jnp.int32),
        grid_spec=pltpu.PrefetchScalarGridSpec(
            num_scalar_prefetch=0,
            grid=(ncores,),
            in_specs=[pl.BlockSpec((3, bb, n), lambda i: (0, i, 0))],
            out_specs=pl.BlockSpec((bb, n_new), lambda i: (i, 0)),
            scratch_shapes=[pltpu.VMEM((hb, n), jnp.float32),
                            pltpu.VMEM((hb, n), jnp.float32),
                            pltpu.VMEM((hb, n_new), jnp.int32),
                            pltpu.VMEM((hb, n_new), jnp.int32)],
        ),
        compiler_params=pltpu.CompilerParams(
            dimension_semantics=("parallel",), vmem_limit_bytes=_VMEM_LIMIT),
    )(pc)
    return jnp.sort(sel, axis=-1)


def _knn_kernel(ps_ref, pt_ref, idx_ref, *, n, k):
    ps = ps_ref[0]
    pt = pt_ref[0]
    tq = ps.shape[0]
    cross = jnp.dot(ps, pt, preferred_element_type=jnp.float32)
    ps2 = jnp.sum(ps * ps, axis=-1, keepdims=True)
    p2 = jnp.sum(pt * pt, axis=0, keepdims=True)
    d2 = ps2 + p2 - 2.0 * cross

    colf = lax.broadcasted_iota(jnp.int32, (tq, n), 1).astype(jnp.float32)
    lanek = lax.broadcasted_iota(jnp.int32, (tq, k), 1)
    big = jnp.float32(3e9)
    acc = jnp.zeros((tq, k), jnp.float32)
    for t in range(k):
        m = jnp.min(d2, axis=-1, keepdims=True)
        sel = jnp.min(jnp.where(d2 == m, colf, big), axis=-1, keepdims=True)
        acc = jnp.where(lanek == t, sel, acc)
        d2 = jnp.where(colf == sel, jnp.inf, d2)
    idx_ref[0] = acc.astype(jnp.int32)


def _knn(p_sub, p, k):
    B, n_new, _ = p_sub.shape
    _, n, _ = p.shape
    tq = min(256, n_new)
    ps = jnp.pad(p_sub, ((0, 0), (0, 0), (0, 5)))
    pt = jnp.pad(jnp.swapaxes(p, 1, 2), ((0, 0), (0, 5), (0, 0)))
    return pl.pallas_call(
        functools.partial(_knn_kernel, n=n, k=k),
        out_shape=jax.ShapeDtypeStruct((B, n_new, k), jnp.int32),
        grid_spec=pltpu.PrefetchScalarGridSpec(
            num_scalar_prefetch=0,
            grid=(B, n_new // tq),
            in_specs=[
                pl.BlockSpec((1, tq, 8), lambda b, i: (b, i, 0)),
                pl.BlockSpec((1, 8, n), lambda b, i: (b, 0, 0)),
            ],
            out_specs=pl.BlockSpec((1, tq, k), lambda b, i: (b, i, 0)),
        ),
        compiler_params=pltpu.CompilerParams(
            dimension_semantics=("parallel", "parallel"),
            vmem_limit_bytes=_VMEM_LIMIT),
    )(ps, pt)


def _mlp_kernel(x_ref, w_ref, b_ref, y_ref):
    y = jnp.dot(x_ref[...], w_ref[...], preferred_element_type=jnp.float32)
    y_ref[...] = jnp.maximum(y + b_ref[...], 0.0)


def _mlp(x_flat, w, bias, *, tm=2048):
    M, d_in = x_flat.shape
    d_out = w.shape[1]
    tm = min(tm, M)
    b2 = bias.reshape(1, d_out)
    return pl.pallas_call(
        _mlp_kernel,
        out_shape=jax.ShapeDtypeStruct((M, d_out), jnp.float32),
        grid_spec=pltpu.PrefetchScalarGridSpec(
            num_scalar_prefetch=0,
            grid=(M // tm,),
            in_specs=[
                pl.BlockSpec((tm, d_in), lambda i: (i, 0)),
                pl.BlockSpec((d_in, d_out), lambda i: (0, 0)),
                pl.BlockSpec((1, d_out), lambda i: (0, 0)),
            ],
            out_specs=pl.BlockSpec((tm, d_out), lambda i: (i, 0)),
        ),
        compiler_params=pltpu.CompilerParams(
            dimension_semantics=("parallel",), vmem_limit_bytes=_VMEM_LIMIT),
    )(x_flat, w, b2)


def _gpool_kernel(y_ref, idx_ref, z_ref, *, tq, k, unroll):
    def run(u, _):
        for qq in range(unroll):
            q = u * unroll + qq
            rows = [y_ref[idx_ref[0, q, j], 0] for j in range(k)]
            while len(rows) > 1:
                rows = [jnp.maximum(a, b) for a, b in zip(rows[::2], rows[1::2])]
            z_ref[q, 0] = rows[0]
        return 0

    lax.fori_loop(0, tq // unroll, run, 0)


def _gather_pool(y3, idx, k, n, n_new, d_out):
    B = idx.shape[0]
    tq = min(256, n_new)
    nblk = n_new // tq
    return pl.pallas_call(
        functools.partial(_gpool_kernel, tq=tq, k=k, unroll=16 if tq % 16 == 0 else 8),
        out_shape=jax.ShapeDtypeStruct((B * n_new, 1, d_out), jnp.float32),
        grid_spec=pltpu.PrefetchScalarGridSpec(
            num_scalar_prefetch=0,
            grid=(B, nblk),
            in_specs=[
                pl.BlockSpec((n, 1, d_out), lambda b, i: (b, 0, 0)),
                pl.BlockSpec((1, tq, k), lambda b, i: (b, i, 0),
                             memory_space=pltpu.SMEM),
            ],
            out_specs=pl.BlockSpec(
                (tq, 1, d_out), lambda b, i, _nblk=nblk: (b * _nblk + i, 0, 0)),
        ),
        compiler_params=pltpu.CompilerParams(
            dimension_semantics=("parallel", "arbitrary"),
            vmem_limit_bytes=_VMEM_LIMIT),
    )(y3, idx)


def kernel(x, p, w, bias, *, factor=2, knn_k=8):
    B, n, d_in = x.shape
    d_out = w.shape[1]
    n_new = -(-n // factor)

    sub_idx = _fps(p, n_new)
    p_sub = jnp.take_along_axis(p, sub_idx[..., None], axis=1)
    knn_idx = _knn(p_sub, p, knn_k)
    y = _mlp(x.reshape(B * n, d_in), w, bias)
    z3 = _gather_pool(y.reshape(B * n, 1, d_out), knn_idx,
                      knn_k, n, n_new, d_out)
    z = z3.reshape(B, n_new, d_out)
    return z, p_sub, knn_idx
